# baseline (speedup 1.0000x reference)
import functools

import jax
import jax.numpy as jnp
from jax import lax
from jax.experimental import pallas as pl
from jax.experimental.pallas import tpu as pltpu
from jax.experimental.pallas import tpu_sc as plsc

MAX_SEQ = 8192
EMB_W = 1024

_NC = 2
_NS = 16
_NW = _NC * _NS

_B_PER_W = MAX_SEQ // _NW
_CHUNK = 16
_N_CHUNKS = _B_PER_W // _CHUNK
_NBUF = 7


def _gather_body(table_hbm, idx_hbm, out_hbm, idx_v, *scratch):
    bufs = scratch[:_NBUF]
    gsems = scratch[_NBUF:2 * _NBUF]
    ssems = scratch[2 * _NBUF:3 * _NBUF]

    wid = lax.axis_index("s") * _NC + lax.axis_index("c")
    base = wid * _B_PER_W
    pltpu.sync_copy(
        idx_hbm.at[pl.ds(base, _CHUNK)], idx_v.at[pl.ds(0, _CHUNK)]
    )

    def chunk_idx(i):
        return idx_v[pl.ds(i * _CHUNK, _CHUNK)]

    gp = [None] * _NBUF
    sp = [None] * _NBUF
    gp[0] = pltpu.async_copy(table_hbm.at[chunk_idx(0)], bufs[0], gsems[0])
    pltpu.sync_copy(
        idx_hbm.at[pl.ds(base + _CHUNK, _B_PER_W - _CHUNK)],
        idx_v.at[pl.ds(_CHUNK, _B_PER_W - _CHUNK)],
    )
    for i in range(1, min(_NBUF, _N_CHUNKS)):
        gp[i] = pltpu.async_copy(table_hbm.at[chunk_idx(i)], bufs[i], gsems[i])
    for i in range(_N_CHUNKS):
        b = i % _NBUF
        gp[b].wait()
        sp[b] = pltpu.async_copy(
            bufs[b], out_hbm.at[pl.ds(base + i * _CHUNK, _CHUNK)], ssems[b]
        )
        j = i + _NBUF
        if j < _N_CHUNKS:
            sp[b].wait()
            gp[b] = pltpu.async_copy(
                table_hbm.at[chunk_idx(j)], bufs[b], gsems[b]
            )
    for b in range(_NBUF):
        if sp[b] is not None:
            sp[b].wait()


@jax.jit
def _gather(inputs, embeddings):
    mesh = plsc.VectorSubcoreMesh(core_axis_name="c", subcore_axis_name="s")
    run = functools.partial(
        pl.kernel,
        mesh=mesh,
        out_type=jax.ShapeDtypeStruct((MAX_SEQ, EMB_W), jnp.float32),
        scratch_types=[pltpu.VMEM((_B_PER_W,), jnp.int32)]
        + [pltpu.VMEM((_CHUNK, EMB_W), jnp.float32) for _ in range(_NBUF)]
        + [pltpu.SemaphoreType.DMA for _ in range(2 * _NBUF)],
    )(_gather_body)
    return run(embeddings, inputs)


def kernel(inputs, embeddings):
    out = _gather(inputs.astype(jnp.int32), embeddings)
    return jnp.expand_dims(out, 0)


# speedup vs baseline: 1.0128x; 1.0128x over previous
import functools

import jax
import jax.numpy as jnp
from jax import lax
from jax.experimental import pallas as pl
from jax.experimental.pallas import tpu as pltpu
from jax.experimental.pallas import tpu_sc as plsc

MAX_SEQ = 8192
EMB_W = 1024

_NC = 2
_NS = 16
_NW = _NC * _NS

_B_PER_W = MAX_SEQ // _NW
_CHUNK = 16
_N_CHUNKS = _B_PER_W // _CHUNK
_NBUF = 7


def _gather_body(table_hbm, idx_hbm, out_hbm, idx_v, *scratch):
    bufs = scratch[:_NBUF]
    gsems = scratch[_NBUF:2 * _NBUF]
    ssems = scratch[2 * _NBUF:3 * _NBUF]

    wid = lax.axis_index("s") * _NC + lax.axis_index("c")
    base = wid * _B_PER_W
    pltpu.sync_copy(idx_hbm.at[pl.ds(base, _B_PER_W)], idx_v)

    def chunk_idx(i):
        return idx_v[pl.ds(i * _CHUNK, _CHUNK)]

    gp = [None] * _NBUF
    sp = [None] * _NBUF
    for i in range(min(_NBUF, _N_CHUNKS)):
        gp[i] = pltpu.async_copy(table_hbm.at[chunk_idx(i)], bufs[i], gsems[i])
    for i in range(_N_CHUNKS):
        b = i % _NBUF
        gp[b].wait()
        sp[b] = pltpu.async_copy(
            bufs[b], out_hbm.at[pl.ds(base + i * _CHUNK, _CHUNK)], ssems[b]
        )
        j = i + _NBUF
        if j < _N_CHUNKS:
            sp[b].wait()
            gp[b] = pltpu.async_copy(
                table_hbm.at[chunk_idx(j)], bufs[b], gsems[b]
            )
    for b in range(_NBUF):
        if sp[b] is not None:
            sp[b].wait()


@jax.jit
def _gather(inputs, embeddings):
    mesh = plsc.VectorSubcoreMesh(core_axis_name="c", subcore_axis_name="s")
    run = functools.partial(
        pl.kernel,
        mesh=mesh,
        out_type=jax.ShapeDtypeStruct((MAX_SEQ, EMB_W), jnp.float32),
        scratch_types=[pltpu.VMEM((_B_PER_W,), jnp.int32)]
        + [pltpu.VMEM((_CHUNK, EMB_W), jnp.float32) for _ in range(_NBUF)]
        + [pltpu.SemaphoreType.DMA for _ in range(2 * _NBUF)],
    )(_gather_body)
    return run(embeddings, inputs)


def kernel(inputs, embeddings):
    out = _gather(inputs.astype(jnp.int32), embeddings)
    return jnp.expand_dims(out, 0)


# speedup vs baseline: 1.0136x; 1.0008x over previous
import functools

import jax
import jax.numpy as jnp
from jax import lax
from jax.experimental import pallas as pl
from jax.experimental.pallas import tpu as pltpu
from jax.experimental.pallas import tpu_sc as plsc

MAX_SEQ = 8192
EMB_W = 1024

_NC = 2
_NS = 16
_NW = _NC * _NS

_B_PER_W = MAX_SEQ // _NW
_CHUNK = 16
_N_CHUNKS = _B_PER_W // _CHUNK
_NBUF = 7


def _gather_body(table_hbm, idx_hbm, out_hbm, idx_v, *scratch):
    bufs = scratch[:_NBUF]
    gsems = scratch[_NBUF:2 * _NBUF]
    ssems = scratch[2 * _NBUF:3 * _NBUF]

    wid = lax.axis_index("s") * _NC + lax.axis_index("c")
    base = wid * _B_PER_W
    pltpu.sync_copy(idx_hbm.at[pl.ds(base, _B_PER_W)], idx_v)

    def chunk_idx(i):
        return idx_v[pl.ds(i * _CHUNK, _CHUNK)]

    gdepth = _NBUF - 3
    gp = [None] * _NBUF
    sp = [None] * _NBUF
    for i in range(min(gdepth, _N_CHUNKS)):
        gp[i % _NBUF] = pltpu.async_copy(
            table_hbm.at[chunk_idx(i)], bufs[i % _NBUF], gsems[i % _NBUF]
        )
    for i in range(_N_CHUNKS):
        b = i % _NBUF
        gp[b].wait()
        sp[b] = pltpu.async_copy(
            bufs[b], out_hbm.at[pl.ds(base + i * _CHUNK, _CHUNK)], ssems[b]
        )
        j = i + gdepth
        if j < _N_CHUNKS:
            bj = j % _NBUF
            if sp[bj] is not None:
                sp[bj].wait()
            gp[bj] = pltpu.async_copy(
                table_hbm.at[chunk_idx(j)], bufs[bj], gsems[bj]
            )
    for b in range(_NBUF):
        if sp[b] is not None:
            sp[b].wait()


@jax.jit
def _gather(inputs, embeddings):
    mesh = plsc.VectorSubcoreMesh(core_axis_name="c", subcore_axis_name="s")
    run = functools.partial(
        pl.kernel,
        mesh=mesh,
        out_type=jax.ShapeDtypeStruct((MAX_SEQ, EMB_W), jnp.float32),
        scratch_types=[pltpu.VMEM((_B_PER_W,), jnp.int32)]
        + [pltpu.VMEM((_CHUNK, EMB_W), jnp.float32) for _ in range(_NBUF)]
        + [pltpu.SemaphoreType.DMA for _ in range(2 * _NBUF)],
    )(_gather_body)
    return run(embeddings, inputs)


def kernel(inputs, embeddings):
    out = _gather(inputs.astype(jnp.int32), embeddings)
    return jnp.expand_dims(out, 0)


# speedup vs baseline: 1.0226x; 1.0089x over previous
import functools

import jax
import jax.numpy as jnp
from jax import lax
from jax.experimental import pallas as pl
from jax.experimental.pallas import tpu as pltpu
from jax.experimental.pallas import tpu_sc as plsc

MAX_SEQ = 8192
EMB_W = 1024

_NC = 2
_NS = 16
_NW = _NC * _NS

_B_PER_W = MAX_SEQ // _NW
_CHUNK = 16
_N_CHUNKS = _B_PER_W // _CHUNK
_NBUF = 6


def _gather_body(table_hbm, idx_hbm, out_hbm, idx_v, *scratch):
    bufs = scratch[:_NBUF]
    gsems = scratch[_NBUF:2 * _NBUF]
    ssems = scratch[2 * _NBUF:3 * _NBUF]

    wid = lax.axis_index("s") * _NC + lax.axis_index("c")
    base = wid * _B_PER_W
    pltpu.sync_copy(idx_hbm.at[pl.ds(base, _B_PER_W)], idx_v)

    def chunk_idx(i):
        return idx_v[pl.ds(i * _CHUNK, _CHUNK)]

    gp = [None] * _NBUF
    sp = [None] * _NBUF
    for i in range(min(_NBUF, _N_CHUNKS)):
        gp[i] = pltpu.async_copy(table_hbm.at[chunk_idx(i)], bufs[i], gsems[i])
    for i in range(_N_CHUNKS):
        b = i % _NBUF
        gp[b].wait()
        sp[b] = pltpu.async_copy(
            bufs[b], out_hbm.at[pl.ds(base + i * _CHUNK, _CHUNK)], ssems[b]
        )
        j = i + _NBUF
        if j < _N_CHUNKS:
            sp[b].wait()
            gp[b] = pltpu.async_copy(
                table_hbm.at[chunk_idx(j)], bufs[b], gsems[b]
            )
    for b in range(_NBUF):
        if sp[b] is not None:
            sp[b].wait()


@jax.jit
def _gather(inputs, embeddings):
    mesh = plsc.VectorSubcoreMesh(core_axis_name="c", subcore_axis_name="s")
    run = functools.partial(
        pl.kernel,
        mesh=mesh,
        out_type=jax.ShapeDtypeStruct((MAX_SEQ, EMB_W), jnp.float32),
        scratch_types=[pltpu.VMEM((_B_PER_W,), jnp.int32)]
        + [pltpu.VMEM((_CHUNK, EMB_W), jnp.float32) for _ in range(_NBUF)]
        + [pltpu.SemaphoreType.DMA for _ in range(2 * _NBUF)],
    )(_gather_body)
    return run(embeddings, inputs)


def kernel(inputs, embeddings):
    out = _gather(inputs.astype(jnp.int32), embeddings)
    return jnp.expand_dims(out, 0)
